# baseline (speedup 1.0000x reference)
import functools

import jax
import jax.numpy as jnp
from jax.experimental import pallas as pl
from jax.experimental.pallas import tpu as pltpu


def _fused_kernel(x_ref, bm_ref, w1_ref, b1_ref, w2_ref, b2_ref,
                  wp1_ref, bp1_ref, wp2_ref, bp2_ref, o_ref, hbuf, w2cb,
                  *, E, PH):
    H = w1_ref.shape[1]

    @pl.when(pl.program_id(0) == 0)
    def _build_w2c():
        w2cb[:H] = w2_ref[...].astype(jnp.bfloat16)
        for e in range(E):
            w2cb[pl.ds(H + e * PH, PH)] = wp2_ref[e].astype(jnp.bfloat16)

    x = x_ref[...]
    bm = bm_ref[...].astype(jnp.float32)
    for c in range(H // PH):
        sl = pl.ds(c * PH, PH)
        h = jnp.dot(x, w1_ref[:, sl], preferred_element_type=jnp.float32)
        hbuf[:, sl] = jnp.maximum(h + b1_ref[:, sl], 0.0).astype(jnp.bfloat16)
    for e in range(E):
        he = jnp.dot(x, wp1_ref[e], preferred_element_type=jnp.float32)
        he = jnp.maximum(he + bp1_ref[e], 0.0) * bm[:, e][:, None]
        hbuf[:, pl.ds(H + e * PH, PH)] = he.astype(jnp.bfloat16)
    o = jnp.dot(hbuf[...], w2cb[...], preferred_element_type=jnp.float32)
    o_ref[...] = o + b2_ref[...] + jnp.dot(
        bm, bp2_ref[...], preferred_element_type=jnp.float32)


def kernel(x, in_bitmap, W1, b1, W2, b2, Wp1, bp1, Wp2, bp2):
    N, D = x.shape
    H = W1.shape[1]
    E, _, PH = Wp1.shape
    C = W2.shape[1]
    F = H + E * PH

    BN = 1024
    grid = (N // BN,)
    out = pl.pallas_call(
        functools.partial(_fused_kernel, E=E, PH=PH),
        grid=grid,
        in_specs=[
            pl.BlockSpec((BN, D), lambda i: (i, 0)),
            pl.BlockSpec((BN, E), lambda i: (i, 0)),
            pl.BlockSpec((D, H), lambda i: (0, 0)),
            pl.BlockSpec((1, H), lambda i: (0, 0)),
            pl.BlockSpec((H, C), lambda i: (0, 0)),
            pl.BlockSpec((1, C), lambda i: (0, 0)),
            pl.BlockSpec((E, D, PH), lambda i: (0, 0, 0)),
            pl.BlockSpec((E, PH), lambda i: (0, 0)),
            pl.BlockSpec((E, PH, C), lambda i: (0, 0, 0)),
            pl.BlockSpec((E, C), lambda i: (0, 0)),
        ],
        out_specs=pl.BlockSpec((BN, C), lambda i: (i, 0)),
        out_shape=jax.ShapeDtypeStruct((N, C), jnp.float32),
        scratch_shapes=[
            pltpu.VMEM((BN, F), jnp.bfloat16),
            pltpu.VMEM((F, C), jnp.bfloat16),
        ],
        compiler_params=pltpu.CompilerParams(
            dimension_semantics=("arbitrary",),
        ),
    )(x, in_bitmap, W1, b1.reshape(1, H), W2, b2.reshape(1, C),
      Wp1, bp1, Wp2, bp2)
    return out


# speedup vs baseline: 1.0252x; 1.0252x over previous
import functools

import jax
import jax.numpy as jnp
from jax.experimental import pallas as pl
from jax.experimental.pallas import tpu as pltpu


def _fused_kernel(x_ref, bm_ref, w1_ref, b1_ref, w2c_ref, b2_ref,
                  wp1_ref, bp1_ref, bp2_ref, o_ref, hbuf, *, E, PH):
    x = x_ref[...]
    bm = bm_ref[...]
    H = w1_ref.shape[1]
    for c in range(H // PH):
        sl = pl.ds(c * PH, PH)
        h = jnp.dot(x, w1_ref[:, sl], preferred_element_type=jnp.float32)
        hbuf[:, sl] = jnp.maximum(h + b1_ref[:, sl], 0.0).astype(jnp.bfloat16)
    for e in range(E):
        he = jnp.dot(x, wp1_ref[e], preferred_element_type=jnp.float32)
        he = jnp.maximum(he + bp1_ref[e], 0.0) * bm[:, e][:, None]
        hbuf[:, pl.ds(H + e * PH, PH)] = he.astype(jnp.bfloat16)
    o = jnp.dot(hbuf[...], w2c_ref[...], preferred_element_type=jnp.float32)
    o_ref[...] = o + b2_ref[...] + jnp.dot(
        bm, bp2_ref[...], preferred_element_type=jnp.float32)


def kernel(x, in_bitmap, W1, b1, W2, b2, Wp1, bp1, Wp2, bp2):
    N, D = x.shape
    H = W1.shape[1]
    E, _, PH = Wp1.shape
    C = W2.shape[1]
    F = H + E * PH

    bm = in_bitmap.astype(jnp.float32)
    W2c = jnp.concatenate([W2, Wp2.reshape(E * PH, C)],
                          axis=0).astype(jnp.bfloat16)

    BN = 1024
    grid = (N // BN,)
    out = pl.pallas_call(
        functools.partial(_fused_kernel, E=E, PH=PH),
        grid=grid,
        in_specs=[
            pl.BlockSpec((BN, D), lambda i: (i, 0)),
            pl.BlockSpec((BN, E), lambda i: (i, 0)),
            pl.BlockSpec((D, H), lambda i: (0, 0)),
            pl.BlockSpec((1, H), lambda i: (0, 0)),
            pl.BlockSpec((F, C), lambda i: (0, 0)),
            pl.BlockSpec((1, C), lambda i: (0, 0)),
            pl.BlockSpec((E, D, PH), lambda i: (0, 0, 0)),
            pl.BlockSpec((E, PH), lambda i: (0, 0)),
            pl.BlockSpec((E, C), lambda i: (0, 0)),
        ],
        out_specs=pl.BlockSpec((BN, C), lambda i: (i, 0)),
        out_shape=jax.ShapeDtypeStruct((N, C), jnp.float32),
        scratch_shapes=[
            pltpu.VMEM((BN, F), jnp.bfloat16),
        ],
        compiler_params=pltpu.CompilerParams(
            dimension_semantics=("arbitrary",),
        ),
    )(x, bm, W1, b1.reshape(1, H), W2c, b2.reshape(1, C), Wp1, bp1, bp2)
    return out
